# baseline (speedup 1.0000x reference)
import functools

import jax
import jax.numpy as jnp
from jax import lax
from jax.experimental import pallas as pl
from jax.experimental.pallas import tpu as pltpu
from jax.experimental.pallas import tpu_sc as plsc

_HIDDEN = 1024
_LANES = 16
_G = _HIDDEN // _LANES
_NC = 2
_NS = 16
_NW = _NC * _NS
_C = 32
_EPS = 1e-12


_GATHER_DNUMS = lax.GatherDimensionNumbers(
    offset_dims=(), collapsed_slice_dims=(0,), start_index_map=(0,))


def _perm16(v, perm):
    return lax.gather(v, perm.reshape(_LANES, 1), _GATHER_DNUMS,
                      slice_sizes=(1,),
                      mode=lax.GatherScatterMode.PROMISE_IN_BOUNDS)


def _splat_sum(v, lane_iota):
    for k in (1, 2, 4, 8):
        v = v + _perm16(v, lane_iota ^ k)
    return v


def _rsqrt_vec(v):
    i = lax.bitcast_convert_type(v, jnp.int32)
    i = jnp.int32(0x5F3759DF) - (i >> 1)
    y = lax.bitcast_convert_type(i, jnp.float32)
    for _ in range(3):
        y = y * (1.5 - 0.5 * v * y * y)
    return y


def _body(vid_hbm, pid_hbm, tid2_hbm, vocab_hbm, pos_hbm, type_hbm,
          gamma_hbm, beta_hbm, out_hbm,
          cvidx, cpidx, ctidx2, vrow, prow, type_v, gamma_v, beta_v, sem):
    n_tokens = out_hbm.shape[0]
    tpw = n_tokens // _NW
    nchunk = tpw // _C
    wid = lax.axis_index("s") * _NC + lax.axis_index("c")
    base = wid * tpw

    pltpu.sync_copy(type_hbm, type_v)
    pltpu.sync_copy(gamma_hbm, gamma_v)
    pltpu.sync_copy(beta_hbm, beta_v)
    trow0 = pl.multiple_of(base // _LANES, tpw // _LANES)
    pltpu.sync_copy(tid2_hbm.at[pl.ds(trow0, tpw // _LANES)], ctidx2)

    def chunk_body(c, carry):
        off = pl.multiple_of(base + c * _C, _C)
        pltpu.sync_copy(vid_hbm.at[pl.ds(off, _C)], cvidx)
        pltpu.sync_copy(pid_hbm.at[pl.ds(off, _C)], cpidx)
        cp_v = pltpu.async_copy(vocab_hbm.at[cvidx], vrow, sem)
        cp_p = pltpu.async_copy(pos_hbm.at[cpidx], prow, sem)
        cp_v.wait()
        cp_p.wait()

        def tok_body(t, tc):
            tv16 = ctidx2[c * (_C // _LANES) + t // _LANES, pl.ds(0, _LANES)]
            lane = t % _LANES
            lane_iota = lax.broadcasted_iota(jnp.int32, (_LANES,), 0)
            tvf = jnp.where(lane_iota == lane, tv16.astype(jnp.float32),
                            jnp.zeros((_LANES,), jnp.float32))
            tm = _splat_sum(tvf, lane_iota) != 0.0
            s = jnp.zeros((_LANES,), jnp.float32)
            q = jnp.zeros((_LANES,), jnp.float32)
            for g in range(_G):
                sl = pl.ds(g * _LANES, _LANES)
                x = (vrow[t, sl] + prow[t, sl]
                     + jnp.where(tm, type_v[1, sl], type_v[0, sl]))
                vrow[t, sl] = x
                s = s + x
                q = q + x * x
            mb = _splat_sum(s, lane_iota) * (1.0 / _HIDDEN)
            var = _splat_sum(q, lane_iota) * (1.0 / _HIDDEN) - mb * mb
            inv = _rsqrt_vec(var + _EPS)
            for g in range(_G):
                sl = pl.ds(g * _LANES, _LANES)
                y = (vrow[t, sl] - mb) * inv * gamma_v[sl] + beta_v[sl]
                vrow[t, sl] = y
            return tc

        lax.fori_loop(0, _C, tok_body, 0)
        pltpu.sync_copy(vrow, out_hbm.at[pl.ds(off, _C)])
        return carry

    lax.fori_loop(0, nchunk, chunk_body, 0)


@jax.jit
def kernel(input_ids, position_ids, token_type_ids, vocab_table, pos_table,
           type_table, ln_gamma, ln_beta):
    b, s = input_ids.shape
    n = b * s
    vid = input_ids.reshape(n).astype(jnp.int32)
    pid = position_ids.reshape(n).astype(jnp.int32)
    tid = token_type_ids.reshape(n // _LANES, _LANES).astype(jnp.int32)

    run = pl.kernel(
        _body,
        out_type=jax.ShapeDtypeStruct((n, _HIDDEN), jnp.float32),
        mesh=plsc.VectorSubcoreMesh(core_axis_name="c", subcore_axis_name="s"),
        scratch_types=[
            pltpu.VMEM((_C,), jnp.int32),
            pltpu.VMEM((_C,), jnp.int32),
            pltpu.VMEM((256 // _LANES, _LANES), jnp.int32),
            pltpu.VMEM((_C, _HIDDEN), jnp.float32),
            pltpu.VMEM((_C, _HIDDEN), jnp.float32),
            pltpu.VMEM((2, _HIDDEN), jnp.float32),
            pltpu.VMEM((_HIDDEN,), jnp.float32),
            pltpu.VMEM((_HIDDEN,), jnp.float32),
            pltpu.SemaphoreType.DMA,
        ],
    )
    out = run(vid, pid, tid, vocab_table, pos_table, type_table,
              ln_gamma, ln_beta)
    return out.reshape(b, s, _HIDDEN)
